# Initial kernel scaffold; baseline (speedup 1.0000x reference)
import jax
import jax.numpy as jnp
from jax.experimental import pallas as pl


def kernel(x, modality, w_gates, W1, b1, W2, b2, Wout, bout):
    raise NotImplementedError("write your pallas kernel here")


# speedup vs baseline: 2.2989x; 2.2989x over previous
import jax
import jax.numpy as jnp
from jax.experimental import pallas as pl

IN_DIM = 1024
OUT_DIM = 1000
PAD_OUT = 1024
NUM_EXPERT = 16
TOP_K = 12
HIDDEN = IN_DIM // 4
N_TOK = 2048
BM = 256

_MM_DTYPE = jnp.bfloat16


def _moe_kernel(x_ref, wg_ref, w1_ref, b1_ref, w2_ref, b2_ref, wout_ref,
                bout_ref, y_ref, gates_ref, load_ref):
    xf = x_ref[...]
    xb = xf.astype(_MM_DTYPE)

    logits = jnp.dot(xb, wg_ref[...], preferred_element_type=jnp.float32)

    lane = jax.lax.broadcasted_iota(jnp.int32, (BM, NUM_EXPERT), 1)
    ranks = jnp.zeros((BM, NUM_EXPERT), jnp.float32)
    for ep in range(NUM_EXPERT):
        col = logits[:, ep:ep + 1]
        beats = (col > logits) | ((col == logits) & (ep < lane))
        ranks = ranks + beats.astype(jnp.float32)
    mask = ranks < float(TOP_K)

    rowmax = jnp.max(logits, axis=1, keepdims=True)
    ex = jnp.where(mask, jnp.exp(logits - rowmax), 0.0)
    gates = ex / jnp.sum(ex, axis=1, keepdims=True)
    gates_ref[...] = gates

    part = jnp.sum((gates > 0.0).astype(jnp.float32), axis=0, keepdims=True)
    i = pl.program_id(0)

    @pl.when(i == 0)
    def _init():
        load_ref[...] = part

    @pl.when(i != 0)
    def _acc():
        load_ref[...] += part

    acc = jnp.dot(gates.astype(_MM_DTYPE), b2_ref[...].astype(_MM_DTYPE),
                  preferred_element_type=jnp.float32)
    for e in range(NUM_EXPERT):
        h = jnp.dot(xb, w1_ref[e], preferred_element_type=jnp.float32)
        h = jnp.maximum(h + b1_ref[e:e + 1, :], 0.0)
        hs = (h * gates[:, e:e + 1]).astype(_MM_DTYPE)
        acc = acc + jnp.dot(hs, w2_ref[e], preferred_element_type=jnp.float32)

    y = jnp.maximum(acc, 0.0) + xf
    out = jnp.dot(y.astype(_MM_DTYPE), wout_ref[...],
                  preferred_element_type=jnp.float32)
    y_ref[...] = out + bout_ref[...]


def kernel(x, modality, w_gates, W1, b1, W2, b2, Wout, bout):
    wg = w_gates[modality].astype(_MM_DTYPE)
    w1 = W1.astype(_MM_DTYPE)
    w2 = W2.astype(_MM_DTYPE)
    wout = jnp.pad(Wout, ((0, 0), (0, PAD_OUT - OUT_DIM))).astype(_MM_DTYPE)
    bout_p = jnp.pad(bout, (0, PAD_OUT - OUT_DIM)).reshape(1, PAD_OUT)

    y_pad, gates, load = pl.pallas_call(
        _moe_kernel,
        grid=(N_TOK // BM,),
        in_specs=[
            pl.BlockSpec((BM, IN_DIM), lambda i: (i, 0)),
            pl.BlockSpec((IN_DIM, NUM_EXPERT), lambda i: (0, 0)),
            pl.BlockSpec((NUM_EXPERT, IN_DIM, HIDDEN), lambda i: (0, 0, 0)),
            pl.BlockSpec((NUM_EXPERT, HIDDEN), lambda i: (0, 0)),
            pl.BlockSpec((NUM_EXPERT, HIDDEN, IN_DIM), lambda i: (0, 0, 0)),
            pl.BlockSpec((NUM_EXPERT, IN_DIM), lambda i: (0, 0)),
            pl.BlockSpec((IN_DIM, PAD_OUT), lambda i: (0, 0)),
            pl.BlockSpec((1, PAD_OUT), lambda i: (0, 0)),
        ],
        out_specs=[
            pl.BlockSpec((BM, PAD_OUT), lambda i: (i, 0)),
            pl.BlockSpec((BM, NUM_EXPERT), lambda i: (i, 0)),
            pl.BlockSpec((1, NUM_EXPERT), lambda i: (0, 0)),
        ],
        out_shape=[
            jax.ShapeDtypeStruct((N_TOK, PAD_OUT), jnp.float32),
            jax.ShapeDtypeStruct((N_TOK, NUM_EXPERT), jnp.float32),
            jax.ShapeDtypeStruct((1, NUM_EXPERT), jnp.float32),
        ],
    )(x, wg, w1, b1, w2, b2, wout, bout_p)

    return (y_pad[:, :OUT_DIM], gates, jnp.reshape(load, (NUM_EXPERT,)))
